# Initial kernel scaffold; baseline (speedup 1.0000x reference)
import jax
import jax.numpy as jnp
from jax.experimental import pallas as pl


def kernel(signal, edge_index, W0, b0, W1, b1, W2, b2, W3, b3, Wm1, bm1, Wm2, bm2):
    raise NotImplementedError("write your pallas kernel here")


# speedup vs baseline: 5.4089x; 5.4089x over previous
import jax
import jax.numpy as jnp
from jax import lax
from jax.experimental import pallas as pl
from jax.experimental.pallas import tpu as pltpu
from jax.experimental.pallas import tpu_sc as plsc

N = 10000
E = 320000
NPAD = 10240
NC, NS = 2, 16
NW = NC * NS
CK = 128
C = 79
EW = C * CK
EPAD = NW * EW
STRIPE = NPAD // NS
RB = 640
G = NPAD // RB
D = 128

_MESH = plsc.VectorSubcoreMesh(
    core_axis_name="c", subcore_axis_name="s", num_cores=NC, num_subcores=NS)


def _spmm_body(u_hbm, src_hbm, dst_hbm, z_hbm, out_hbm, srcv, dstv, rows, acc, sem):
    c = lax.axis_index("c")
    s = lax.axis_index("s")
    w = s * NC + c
    pltpu.sync_copy(z_hbm, acc.at[pl.ds(s * STRIPE, STRIPE), :])
    pltpu.sync_copy(src_hbm.at[w], srcv)
    pltpu.sync_copy(dst_hbm.at[w], dstv)
    plsc.subcore_barrier()

    def chunk(i, carry):
        pltpu.async_copy(u_hbm.at[srcv.at[i]], rows, sem).wait()
        pltpu.sync_copy(rows, acc.at[dstv.at[i]], add=True)
        return carry

    lax.fori_loop(0, C, chunk, 0)
    plsc.subcore_barrier()
    pltpu.sync_copy(acc.at[pl.ds(s * STRIPE, STRIPE), :],
                    out_hbm.at[c, pl.ds(s * STRIPE, STRIPE), :])


_spmm_kernel = pl.kernel(
    _spmm_body,
    out_type=jax.ShapeDtypeStruct((NC, NPAD, D), jnp.float32),
    mesh=_MESH,
    scratch_types=[
        pltpu.VMEM((C, CK), jnp.int32),
        pltpu.VMEM((C, CK), jnp.int32),
        pltpu.VMEM((CK, D), jnp.float32),
        pltpu.VMEM_SHARED((NPAD, D), jnp.float32),
        pltpu.SemaphoreType.DMA,
    ],
)


def _deg_body(dst_hbm, ones_hbm, z_hbm, out_hbm, dstv, ones_v, acc):
    c = lax.axis_index("c")
    s = lax.axis_index("s")
    w = s * NC + c
    pltpu.sync_copy(z_hbm, acc.at[pl.ds(s * STRIPE, STRIPE), :])
    pltpu.sync_copy(dst_hbm.at[w], dstv)
    pltpu.sync_copy(ones_hbm, ones_v)
    plsc.subcore_barrier()

    def chunk(i, carry):
        pltpu.sync_copy(ones_v, acc.at[dstv.at[i]], add=True)
        return carry

    lax.fori_loop(0, C, chunk, 0)
    plsc.subcore_barrier()
    pltpu.sync_copy(acc.at[pl.ds(s * STRIPE, STRIPE), :],
                    out_hbm.at[c, pl.ds(s * STRIPE, STRIPE), :])


_deg_kernel = pl.kernel(
    _deg_body,
    out_type=jax.ShapeDtypeStruct((NC, NPAD, D), jnp.float32),
    mesh=_MESH,
    scratch_types=[
        pltpu.VMEM((C, CK), jnp.int32),
        pltpu.VMEM((CK, D), jnp.float32),
        pltpu.VMEM_SHARED((NPAD, D), jnp.float32),
    ],
)


def _dinv_body(degp_ref, x_ref, dinv_ref, u0_ref):
    deg = degp_ref[0, :, 0:1] + degp_ref[1, :, 0:1]
    di = lax.rsqrt(jnp.maximum(deg, 1.0))
    dinv_ref[...] = di
    u0_ref[...] = di * x_ref[...]


def _dinv_call(degp, x_pad):
    return pl.pallas_call(
        _dinv_body,
        grid=(G,),
        in_specs=[
            pl.BlockSpec((NC, RB, D), lambda i: (0, i, 0)),
            pl.BlockSpec((RB, 128), lambda i: (i, 0)),
        ],
        out_specs=[
            pl.BlockSpec((RB, 1), lambda i: (i, 0)),
            pl.BlockSpec((RB, 128), lambda i: (i, 0)),
        ],
        out_shape=[
            jax.ShapeDtypeStruct((NPAD, 1), jnp.float32),
            jax.ShapeDtypeStruct((NPAD, 128), jnp.float32),
        ],
    )(degp, x_pad)


def _tca_body(din, sp_ref, dinv_ref, z1_ref, u1_ref):
    di = dinv_ref[...]
    z1 = -(di * (sp_ref[0][:, :din] + sp_ref[1][:, :din]))
    z1_ref[...] = z1
    u1 = di * z1
    if din < D:
        u1 = jnp.concatenate([u1, jnp.zeros((RB, D - din), jnp.float32)], axis=1)
    u1_ref[...] = u1


def _tca_call(sp, dinv, din):
    return pl.pallas_call(
        lambda *refs: _tca_body(din, *refs),
        grid=(G,),
        in_specs=[
            pl.BlockSpec((NC, RB, D), lambda i: (0, i, 0)),
            pl.BlockSpec((RB, 1), lambda i: (i, 0)),
        ],
        out_specs=[
            pl.BlockSpec((RB, din), lambda i: (i, 0)),
            pl.BlockSpec((RB, D), lambda i: (i, 0)),
        ],
        out_shape=[
            jax.ShapeDtypeStruct((NPAD, din), jnp.float32),
            jax.ShapeDtypeStruct((NPAD, D), jnp.float32),
        ],
    )(sp, dinv)


def _tcb_body(din, dout, sp_ref, dinv_ref, z0_ref, z1_ref, w_ref, b_ref,
              h_ref, un_ref):
    di = dinv_ref[...]
    z0 = z0_ref[...]
    z2 = -2.0 * (di * (sp_ref[0][:, :din] + sp_ref[1][:, :din])) - z0
    h = (jnp.dot(z0, w_ref[0], preferred_element_type=jnp.float32)
         + jnp.dot(z1_ref[...], w_ref[1], preferred_element_type=jnp.float32)
         + jnp.dot(z2, w_ref[2], preferred_element_type=jnp.float32)
         + b_ref[...])
    h_ref[...] = h
    un = di * h
    if dout < D:
        un = jnp.concatenate([un, jnp.zeros((RB, D - dout), jnp.float32)], axis=1)
    un_ref[...] = un


def _tcb_call(sp, dinv, z0, z1, W, b2d, din, dout):
    return pl.pallas_call(
        lambda *refs: _tcb_body(din, dout, *refs),
        grid=(G,),
        in_specs=[
            pl.BlockSpec((NC, RB, D), lambda i: (0, i, 0)),
            pl.BlockSpec((RB, 1), lambda i: (i, 0)),
            pl.BlockSpec((RB, din), lambda i: (i, 0)),
            pl.BlockSpec((RB, din), lambda i: (i, 0)),
            pl.BlockSpec((3, din, dout), lambda i: (0, 0, 0)),
            pl.BlockSpec((1, dout), lambda i: (0, 0)),
        ],
        out_specs=[
            pl.BlockSpec((RB, dout), lambda i: (i, 0)),
            pl.BlockSpec((RB, D), lambda i: (i, 0)),
        ],
        out_shape=[
            jax.ShapeDtypeStruct((NPAD, dout), jnp.float32),
            jax.ShapeDtypeStruct((NPAD, D), jnp.float32),
        ],
    )(sp, dinv, z0, z1, W, b2d)


def _tcb3_body(din, dout, sp_ref, dinv_ref, z0_ref, z1_ref, w_ref, b_ref,
               wm1_ref, bm1_ref, wm2_ref, bm2_ref, out_ref, hg_acc):
    i = pl.program_id(0)
    di = dinv_ref[...]
    z0 = z0_ref[...]
    z2 = -2.0 * (di * (sp_ref[0][:, :din] + sp_ref[1][:, :din])) - z0
    h = (jnp.dot(z0, w_ref[0], preferred_element_type=jnp.float32)
         + jnp.dot(z1_ref[...], w_ref[1], preferred_element_type=jnp.float32)
         + jnp.dot(z2, w_ref[2], preferred_element_type=jnp.float32)
         + b_ref[...])
    rows = lax.broadcasted_iota(jnp.int32, (RB, 1), 0) + i * RB
    h = jnp.where(rows < N, h, 0.0)
    psum = jnp.sum(h, axis=0, keepdims=True)

    @pl.when(i == 0)
    def _():
        hg_acc[...] = psum

    @pl.when(i > 0)
    def _():
        hg_acc[...] = hg_acc[...] + psum

    @pl.when(i == G - 1)
    def _():
        hg = hg_acc[...]
        a = jnp.maximum(
            jnp.dot(hg, wm1_ref[...], preferred_element_type=jnp.float32)
            + bm1_ref[...], 0.0)
        out_ref[...] = (jnp.dot(a, wm2_ref[...], preferred_element_type=jnp.float32)
                        + bm2_ref[...])


def _tcb3_call(sp, dinv, z0, z1, W, b2d, Wm1, bm1_2d, Wm2, bm2_2d, din, dout):
    return pl.pallas_call(
        lambda *refs: _tcb3_body(din, dout, *refs),
        grid=(G,),
        in_specs=[
            pl.BlockSpec((NC, RB, D), lambda i: (0, i, 0)),
            pl.BlockSpec((RB, 1), lambda i: (i, 0)),
            pl.BlockSpec((RB, din), lambda i: (i, 0)),
            pl.BlockSpec((RB, din), lambda i: (i, 0)),
            pl.BlockSpec((3, din, dout), lambda i: (0, 0, 0)),
            pl.BlockSpec((1, dout), lambda i: (0, 0)),
            pl.BlockSpec((dout, dout), lambda i: (0, 0)),
            pl.BlockSpec((1, dout), lambda i: (0, 0)),
            pl.BlockSpec((dout, 10), lambda i: (0, 0)),
            pl.BlockSpec((1, 10), lambda i: (0, 0)),
        ],
        out_specs=pl.BlockSpec((1, 10), lambda i: (0, 0)),
        out_shape=jax.ShapeDtypeStruct((1, 10), jnp.float32),
        scratch_shapes=[pltpu.VMEM((1, dout), jnp.float32)],
    )(sp, dinv, z0, z1, W, b2d, Wm1, bm1_2d, Wm2, bm2_2d)


def kernel(signal, edge_index, W0, b0, W1, b1, W2, b2, W3, b3, Wm1, bm1, Wm2, bm2):
    src = edge_index[0]
    dst = edge_index[1]
    pad = EPAD - E
    src3 = jnp.concatenate([src, jnp.zeros((pad,), jnp.int32)]).reshape(NW, C, CK)
    dst3 = jnp.concatenate([dst, jnp.full((pad,), N, jnp.int32)]).reshape(NW, C, CK)
    x_pad = jnp.concatenate(
        [signal, jnp.zeros((NPAD - N, signal.shape[1]), jnp.float32)], axis=0)

    z128 = jnp.zeros((STRIPE, D), jnp.float32)
    ones128 = jnp.ones((CK, D), jnp.float32)

    degp = _deg_kernel(dst3, ones128, z128)
    dinv, u0 = _dinv_call(degp, x_pad)

    layers = [
        (W0, b0, 128, 64),
        (W1, b1, 64, 64),
        (W2, b2, 64, 128),
        (W3, b3, 128, 256),
    ]
    z0 = x_pad
    u = u0
    for li, (W, b, din, dout) in enumerate(layers):
        sp0 = _spmm_kernel(u, src3, dst3, z128)
        z1, u1 = _tca_call(sp0, dinv, din)
        sp1 = _spmm_kernel(u1, src3, dst3, z128)
        b2d = b.reshape(1, dout)
        if li < 3:
            z0, u = _tcb_call(sp1, dinv, z0, z1, W, b2d, din, dout)
        else:
            out = _tcb3_call(sp1, dinv, z0, z1, W, b2d,
                             Wm1, bm1.reshape(1, dout), Wm2, bm2.reshape(1, 10),
                             din, dout)
    return out
